# baseline (speedup 1.0000x reference)
import jax
import jax.numpy as jnp
from jax.experimental import pallas as pl


def _relu_matmul_bias_kernel(x_ref, w_ref, b_ref, o_ref):
    x = jnp.maximum(x_ref[...], 0.0)
    acc = jax.lax.dot_general(
        x, w_ref[...], (((1,), (0,)), ((), ())),
        preferred_element_type=jnp.float32,
    )
    o_ref[...] = acc + b_ref[...]


def kernel(x_subject, x_region, edge_index_sr, edge_index_rr, edge_attr_sr,
           edge_attr_rr, sage_Wl0, sage_bl0, sage_Wr0, gcn_W0, gcn_b0,
           sage_Wl1, sage_bl1, sage_Wr1, gcn_W1, gcn_b1, lin_W, lin_b):
    m, d = x_subject.shape
    out_dim = lin_W.shape[1]
    bm = 1000
    return pl.pallas_call(
        _relu_matmul_bias_kernel,
        grid=(m // bm,),
        in_specs=[
            pl.BlockSpec((bm, d), lambda i: (i, 0)),
            pl.BlockSpec((d, out_dim), lambda i: (0, 0)),
            pl.BlockSpec((1, out_dim), lambda i: (0, 0)),
        ],
        out_specs=pl.BlockSpec((bm, out_dim), lambda i: (i, 0)),
        out_shape=jax.ShapeDtypeStruct((m, out_dim), jnp.float32),
    )(x_subject, lin_W, lin_b.reshape(1, out_dim))


# speedup vs baseline: 1.1597x; 1.1597x over previous
import jax
import jax.numpy as jnp
from jax.experimental import pallas as pl


def _relu_matmul_bias_kernel(x_ref, w_ref, b_ref, o_ref):
    x = jnp.maximum(x_ref[...], 0.0)
    acc = jax.lax.dot_general(
        x, w_ref[...], (((1,), (0,)), ((), ())),
        preferred_element_type=jnp.float32,
    )
    o_ref[...] = acc + b_ref[...]


def kernel(x_subject, x_region, edge_index_sr, edge_index_rr, edge_attr_sr,
           edge_attr_rr, sage_Wl0, sage_bl0, sage_Wr0, gcn_W0, gcn_b0,
           sage_Wl1, sage_bl1, sage_Wr1, gcn_W1, gcn_b1, lin_W, lin_b):
    m, d = x_subject.shape
    out_dim = lin_W.shape[1]
    bm = 2000
    return pl.pallas_call(
        _relu_matmul_bias_kernel,
        grid=(m // bm,),
        in_specs=[
            pl.BlockSpec((bm, d), lambda i: (i, 0)),
            pl.BlockSpec((d, out_dim), lambda i: (0, 0)),
            pl.BlockSpec((1, out_dim), lambda i: (0, 0)),
        ],
        out_specs=pl.BlockSpec((bm, out_dim), lambda i: (i, 0)),
        out_shape=jax.ShapeDtypeStruct((m, out_dim), jnp.float32),
    )(x_subject, lin_W, lin_b.reshape(1, out_dim))


# speedup vs baseline: 1.3119x; 1.1312x over previous
import jax
import jax.numpy as jnp
from jax.experimental import pallas as pl


def _relu_matmul_bias_kernel(x_ref, w_ref, b_ref, o_ref):
    x = jnp.maximum(x_ref[...], 0.0)
    acc = jax.lax.dot_general(
        x, w_ref[...], (((1,), (0,)), ((), ())),
        preferred_element_type=jnp.float32,
    )
    o_ref[...] = acc + b_ref[...]


def kernel(x_subject, x_region, edge_index_sr, edge_index_rr, edge_attr_sr,
           edge_attr_rr, sage_Wl0, sage_bl0, sage_Wr0, gcn_W0, gcn_b0,
           sage_Wl1, sage_bl1, sage_Wr1, gcn_W1, gcn_b1, lin_W, lin_b):
    m, d = x_subject.shape
    out_dim = lin_W.shape[1]
    bm = 10000
    return pl.pallas_call(
        _relu_matmul_bias_kernel,
        grid=(m // bm,),
        in_specs=[
            pl.BlockSpec((bm, d), lambda i: (i, 0)),
            pl.BlockSpec((d, out_dim), lambda i: (0, 0)),
            pl.BlockSpec((1, out_dim), lambda i: (0, 0)),
        ],
        out_specs=pl.BlockSpec((bm, out_dim), lambda i: (i, 0)),
        out_shape=jax.ShapeDtypeStruct((m, out_dim), jnp.float32),
    )(x_subject, lin_W, lin_b.reshape(1, out_dim))


# speedup vs baseline: 1.3157x; 1.0029x over previous
import jax
import jax.numpy as jnp
from jax.experimental import pallas as pl


def _relu_matmul_bias_kernel(x_ref, w_ref, b_ref, o_ref):
    x = jnp.maximum(x_ref[...], 0.0)
    acc = jax.lax.dot_general(
        x, w_ref[...], (((1,), (0,)), ((), ())),
        preferred_element_type=jnp.float32,
    )
    o_ref[...] = acc + b_ref[...]


def kernel(x_subject, x_region, edge_index_sr, edge_index_rr, edge_attr_sr,
           edge_attr_rr, sage_Wl0, sage_bl0, sage_Wr0, gcn_W0, gcn_b0,
           sage_Wl1, sage_bl1, sage_Wr1, gcn_W1, gcn_b1, lin_W, lin_b):
    m, d = x_subject.shape
    out_dim = lin_W.shape[1]
    return pl.pallas_call(
        _relu_matmul_bias_kernel,
        out_shape=jax.ShapeDtypeStruct((m, out_dim), jnp.float32),
    )(x_subject, lin_W, lin_b.reshape(1, out_dim))
